# baseline (speedup 1.0000x reference)
import jax
import jax.numpy as jnp
from jax.experimental import pallas as pl

def _add_block(x_ref, p_ref, o_ref):
    o_ref[...] = x_ref[...] + p_ref[...]

def kernel(x, pos_emb):
    S, D = x.shape
    BS = 2048
    out = pl.pallas_call(
        _add_block,
        grid=(S // BS,),
        in_specs=[
            pl.BlockSpec((BS, D), lambda i: (i, 0)),
            pl.BlockSpec((BS, D), lambda i: (i, 0)),
        ],
        out_specs=pl.BlockSpec((BS, D), lambda i: (i, 0)),
        out_shape=jax.ShapeDtypeStruct((S, D), x.dtype),
    )(x, pos_emb)
    return out[None]
